# baseline (speedup 1.0000x reference)
import functools
import numpy as np
import jax
import jax.numpy as jnp
from jax import lax
from jax.experimental import pallas as pl
from jax.experimental.pallas import tpu as pltpu
from jax.experimental.pallas import tpu_sc as plsc

HID = 256
ND = 3
K = 32
TS = 128
MT = 8192


def _embed_consts():
    per = HID // ND
    half = per // 2
    emb = np.exp(np.arange(half) * -(np.log(10000.0) / (half - 1)))
    F = np.zeros((ND, HID), np.float32)
    ph = np.zeros((HID,), np.float32)
    w = 2 * half
    for i in range(ND):
        F[i, w * i: w * i + half] = emb
        F[i, w * i + half: w * i + 2 * half] = emb
        ph[w * i + half: w * i + 2 * half] = np.pi / 2
    return jnp.asarray(F), jnp.asarray(ph.reshape(1, HID))


def _topk_kernel(si_ref, xs_ref, xst_ref, o_ref):
    N = xs_ref.shape[1]
    b = pl.program_id(0)
    xs = xs_ref[0]
    xst = xst_ref[0]
    si = si_ref[0]
    iota = jax.lax.broadcasted_iota(jnp.int32, (TS, N), 1)

    q = jax.lax.dot_general((iota == si).astype(jnp.float32), xs,
                            (((1,), (0,)), ((), ())),
                            precision=jax.lax.Precision.HIGHEST)
    dist = jnp.zeros((TS, N), jnp.float32)
    for d in range(ND):
        diff = q[:, d:d + 1] - xst[d:d + 1, :]
        dist = dist + diff * diff

    idxs = []
    for _ in range(K):
        idx = jnp.argmin(dist, axis=1).astype(jnp.int32)[:, None]
        dist = jnp.where(iota == idx, jnp.inf, dist)
        idxs.append(idx)
    o_ref[0] = jnp.concatenate(idxs, axis=1) + b * N


def _make_sc_gather(n_pts, tab_len):
    info = plsc.get_sparse_core_info()
    nc, ns, L = info.num_cores, info.num_subcores, info.num_lanes
    nw = nc * ns
    per_w = n_pts // nw
    mesh = plsc.VectorSubcoreMesh(core_axis_name="c", subcore_axis_name="s")

    @functools.partial(
        pl.kernel, mesh=mesh,
        out_type=jax.ShapeDtypeStruct((nw, ND * per_w), jnp.float32),
        scratch_types=[
            pltpu.VMEM((tab_len,), jnp.float32),
            pltpu.VMEM((per_w,), jnp.int32),
            pltpu.VMEM((ND * per_w,), jnp.float32),
        ],
        compiler_params=pltpu.CompilerParams(needs_layout_passes=False),
    )
    def sc_gather(tab_hbm, idx_hbm, out_hbm, tab_v, idx_v, out_v):
        wid = lax.axis_index("s") * nc + lax.axis_index("c")
        base = wid * per_w
        pltpu.sync_copy(tab_hbm, tab_v)
        pltpu.sync_copy(idx_hbm.at[pl.ds(base, per_w)], idx_v)

        def body(i, carry):
            f = jax.lax.iota(jnp.int32, L) + i * L
            j = f // ND
            rows = plsc.load_gather(idx_v, [j])
            src = rows * ND + (f - j * ND)
            out_v[pl.ds(i * L, L)] = plsc.load_gather(tab_v, [src])
            return carry

        lax.fori_loop(0, (ND * per_w) // L, body, 0)
        pltpu.sync_copy(out_v, out_hbm.at[wid])

    return sc_gather, nw, per_w


def _mlp_kernel(pts_ref, win_ref, bin_ref, f_ref, ph_ref,
                w1_ref, b1_ref, w2_ref, b2_ref, o_ref):
    p = pts_ref[...]
    x = (p @ win_ref[...] + bin_ref[...]
         + jnp.sin(p @ f_ref[...] + ph_ref[...]))
    h = jax.nn.gelu(x @ w1_ref[...] + b1_ref[...])
    hm = jnp.mean(h.reshape(MT // K, K, HID), axis=1)
    o_ref[...] = hm @ w2_ref[...] + b2_ref[...]


def kernel(input_pos, supernode_idxs, W_in, b_in, W1, b1, W2, b2):
    B, N, _ = input_pos.shape
    S = supernode_idxs.shape[1]
    nt = S // TS
    M = B * S * K
    si = supernode_idxs.astype(jnp.int32).reshape(B * nt, TS, 1)
    xst = jnp.transpose(input_pos, (0, 2, 1))
    F, ph = _embed_consts()

    idx = pl.pallas_call(
        _topk_kernel,
        grid=(B, nt),
        in_specs=[
            pl.BlockSpec((1, TS, 1), lambda b, j: (b * nt + j, 0, 0)),
            pl.BlockSpec((1, N, ND), lambda b, j: (b, 0, 0)),
            pl.BlockSpec((1, ND, N), lambda b, j: (b, 0, 0)),
        ],
        out_specs=pl.BlockSpec((1, TS, K), lambda b, j: (b * nt + j, 0, 0)),
        out_shape=jax.ShapeDtypeStruct((B * nt, TS, K), jnp.int32),
    )(si, input_pos, xst)
    flat_idx = idx.reshape(M)

    tab = input_pos.reshape(B * N * ND)
    sc_gather, nw, per_w = _make_sc_gather(M, B * N * ND)
    ptsw = sc_gather(tab, flat_idx)
    pts = ptsw.reshape(M, ND)

    out = pl.pallas_call(
        _mlp_kernel,
        grid=(M // MT,),
        in_specs=[
            pl.BlockSpec((MT, ND), lambda j: (j, 0)),
            pl.BlockSpec((ND, HID), lambda j: (0, 0)),
            pl.BlockSpec((1, HID), lambda j: (0, 0)),
            pl.BlockSpec((ND, HID), lambda j: (0, 0)),
            pl.BlockSpec((1, HID), lambda j: (0, 0)),
            pl.BlockSpec((HID, HID), lambda j: (0, 0)),
            pl.BlockSpec((1, HID), lambda j: (0, 0)),
            pl.BlockSpec((HID, HID), lambda j: (0, 0)),
            pl.BlockSpec((1, HID), lambda j: (0, 0)),
        ],
        out_specs=pl.BlockSpec((MT // K, HID), lambda j: (j, 0)),
        out_shape=jax.ShapeDtypeStruct((B * S, HID), jnp.float32),
    )(pts, W_in, b_in.reshape(1, HID), F, ph,
      W1, b1.reshape(1, HID), W2, b2.reshape(1, HID))
    return out.reshape(B, S, HID)


# speedup vs baseline: 1.0581x; 1.0581x over previous
import functools
import numpy as np
import jax
import jax.numpy as jnp
from jax import lax
from jax.experimental import pallas as pl
from jax.experimental.pallas import tpu as pltpu
from jax.experimental.pallas import tpu_sc as plsc

HID = 256
ND = 3
K = 32
TS = 128
MT = 8192


def _embed_consts():
    per = HID // ND
    half = per // 2
    emb = np.exp(np.arange(half) * -(np.log(10000.0) / (half - 1)))
    F = np.zeros((ND, HID), np.float32)
    ph = np.zeros((HID,), np.float32)
    w = 2 * half
    for i in range(ND):
        F[i, w * i: w * i + half] = emb
        F[i, w * i + half: w * i + 2 * half] = emb
        ph[w * i + half: w * i + 2 * half] = np.pi / 2
    return jnp.asarray(F), jnp.asarray(ph.reshape(1, HID))


def _topk_kernel(si_ref, xs_ref, xst_ref, o_ref):
    N = xs_ref.shape[1]
    b = pl.program_id(0)
    xs = xs_ref[0]
    xst = xst_ref[0]
    si = si_ref[0]
    iota = jax.lax.broadcasted_iota(jnp.int32, (TS, N), 1)

    q = jax.lax.dot_general((iota == si).astype(jnp.float32), xs,
                            (((1,), (0,)), ((), ())),
                            precision=jax.lax.Precision.HIGHEST)
    dist = jnp.zeros((TS, N), jnp.float32)
    for d in range(ND):
        diff = q[:, d:d + 1] - xst[d:d + 1, :]
        dist = dist + diff * diff

    idxs = []
    for _ in range(K):
        idx = jnp.argmin(dist, axis=1).astype(jnp.int32)[:, None]
        dist = jnp.where(iota == idx, jnp.inf, dist)
        idxs.append(idx)
    o_ref[0] = jnp.concatenate(idxs, axis=1) + b * N


def _make_sc_gather(n_pts, tab_len):
    info = plsc.get_sparse_core_info()
    nc, ns, L = info.num_cores, info.num_subcores, info.num_lanes
    nw = nc * ns
    per_w = n_pts // nw
    mesh = plsc.VectorSubcoreMesh(core_axis_name="c", subcore_axis_name="s")

    @functools.partial(
        pl.kernel, mesh=mesh,
        out_type=jax.ShapeDtypeStruct((nw, ND * per_w), jnp.float32),
        scratch_types=[
            pltpu.VMEM((tab_len,), jnp.float32),
            pltpu.VMEM((per_w,), jnp.int32),
            pltpu.VMEM((ND * per_w,), jnp.float32),
        ],
        compiler_params=pltpu.CompilerParams(needs_layout_passes=False),
    )
    def sc_gather(tab_hbm, idx_hbm, out_hbm, tab_v, idx_v, out_v):
        wid = lax.axis_index("s") * nc + lax.axis_index("c")
        base = wid * per_w
        pltpu.sync_copy(tab_hbm, tab_v)
        pltpu.sync_copy(idx_hbm.at[pl.ds(base, per_w)], idx_v)

        def body(i, carry):
            rows = idx_v[pl.ds(i * L, L)]
            r3 = rows * ND
            for d in range(ND):
                out_v[pl.ds(d * per_w + i * L, L)] = plsc.load_gather(
                    tab_v, [r3 + d])
            return carry

        lax.fori_loop(0, per_w // L, body, 0)
        pltpu.sync_copy(out_v, out_hbm.at[wid])

    return sc_gather, nw, per_w


def _mlp_kernel(pts_ref, win_ref, bin_ref, f_ref, ph_ref,
                w1_ref, b1_ref, w2_ref, b2_ref, o_ref):
    p = pts_ref[...]
    x = (p @ win_ref[...] + bin_ref[...]
         + jnp.sin(p @ f_ref[...] + ph_ref[...]))
    h = jax.nn.gelu(x @ w1_ref[...] + b1_ref[...])
    hm = jnp.mean(h.reshape(MT // K, K, HID), axis=1)
    o_ref[...] = hm @ w2_ref[...] + b2_ref[...]


def kernel(input_pos, supernode_idxs, W_in, b_in, W1, b1, W2, b2):
    B, N, _ = input_pos.shape
    S = supernode_idxs.shape[1]
    nt = S // TS
    M = B * S * K
    si = supernode_idxs.astype(jnp.int32).reshape(B * nt, TS, 1)
    xst = jnp.transpose(input_pos, (0, 2, 1))
    F, ph = _embed_consts()

    idx = pl.pallas_call(
        _topk_kernel,
        grid=(B, nt),
        in_specs=[
            pl.BlockSpec((1, TS, 1), lambda b, j: (b * nt + j, 0, 0)),
            pl.BlockSpec((1, N, ND), lambda b, j: (b, 0, 0)),
            pl.BlockSpec((1, ND, N), lambda b, j: (b, 0, 0)),
        ],
        out_specs=pl.BlockSpec((1, TS, K), lambda b, j: (b * nt + j, 0, 0)),
        out_shape=jax.ShapeDtypeStruct((B * nt, TS, K), jnp.int32),
    )(si, input_pos, xst)
    flat_idx = idx.reshape(M)

    tab = input_pos.reshape(B * N * ND)
    sc_gather, nw, per_w = _make_sc_gather(M, B * N * ND)
    ptsw = sc_gather(tab, flat_idx)
    pts = jnp.transpose(ptsw.reshape(nw, ND, per_w), (0, 2, 1)).reshape(M, ND)

    out = pl.pallas_call(
        _mlp_kernel,
        grid=(M // MT,),
        in_specs=[
            pl.BlockSpec((MT, ND), lambda j: (j, 0)),
            pl.BlockSpec((ND, HID), lambda j: (0, 0)),
            pl.BlockSpec((1, HID), lambda j: (0, 0)),
            pl.BlockSpec((ND, HID), lambda j: (0, 0)),
            pl.BlockSpec((1, HID), lambda j: (0, 0)),
            pl.BlockSpec((HID, HID), lambda j: (0, 0)),
            pl.BlockSpec((1, HID), lambda j: (0, 0)),
            pl.BlockSpec((HID, HID), lambda j: (0, 0)),
            pl.BlockSpec((1, HID), lambda j: (0, 0)),
        ],
        out_specs=pl.BlockSpec((MT // K, HID), lambda j: (j, 0)),
        out_shape=jax.ShapeDtypeStruct((B * S, HID), jnp.float32),
    )(pts, W_in, b_in.reshape(1, HID), F, ph,
      W1, b1.reshape(1, HID), W2, b2.reshape(1, HID))
    return out.reshape(B, S, HID)


# speedup vs baseline: 1.0599x; 1.0016x over previous
import functools
import numpy as np
import jax
import jax.numpy as jnp
from jax import lax
from jax.experimental import pallas as pl
from jax.experimental.pallas import tpu as pltpu
from jax.experimental.pallas import tpu_sc as plsc

HID = 256
ND = 3
K = 32
TS = 256
MT = 8192


def _embed_consts():
    per = HID // ND
    half = per // 2
    emb = np.exp(np.arange(half) * -(np.log(10000.0) / (half - 1)))
    F = np.zeros((ND, HID), np.float32)
    ph = np.zeros((HID,), np.float32)
    w = 2 * half
    for i in range(ND):
        F[i, w * i: w * i + half] = emb
        F[i, w * i + half: w * i + 2 * half] = emb
        ph[w * i + half: w * i + 2 * half] = np.pi / 2
    return jnp.asarray(F), jnp.asarray(ph.reshape(1, HID))


def _topk_kernel(si_ref, xs_ref, xst_ref, o_ref):
    N = xs_ref.shape[1]
    b = pl.program_id(0)
    xs = xs_ref[0]
    xst = xst_ref[0]
    si = si_ref[0]
    iota = jax.lax.broadcasted_iota(jnp.int32, (TS, N), 1)

    q = jax.lax.dot_general((iota == si).astype(jnp.float32), xs,
                            (((1,), (0,)), ((), ())),
                            precision=jax.lax.Precision.HIGHEST)
    dist = jnp.zeros((TS, N), jnp.float32)
    for d in range(ND):
        diff = q[:, d:d + 1] - xst[d:d + 1, :]
        dist = dist + diff * diff

    idxs = []
    for _ in range(K):
        idx = jnp.argmin(dist, axis=1).astype(jnp.int32)[:, None]
        dist = jnp.where(iota == idx, jnp.inf, dist)
        idxs.append(idx)
    o_ref[0] = jnp.concatenate(idxs, axis=1) + b * N


def _make_sc_gather(n_pts, tab_len):
    info = plsc.get_sparse_core_info()
    nc, ns, L = info.num_cores, info.num_subcores, info.num_lanes
    nw = nc * ns
    per_w = n_pts // nw
    mesh = plsc.VectorSubcoreMesh(core_axis_name="c", subcore_axis_name="s")

    @functools.partial(
        pl.kernel, mesh=mesh,
        out_type=jax.ShapeDtypeStruct((nw, ND * per_w), jnp.float32),
        scratch_types=[
            pltpu.VMEM((tab_len,), jnp.float32),
            pltpu.VMEM((per_w,), jnp.int32),
            pltpu.VMEM((ND * per_w,), jnp.float32),
        ],
        compiler_params=pltpu.CompilerParams(needs_layout_passes=False),
    )
    def sc_gather(tab_hbm, idx_hbm, out_hbm, tab_v, idx_v, out_v):
        wid = lax.axis_index("s") * nc + lax.axis_index("c")
        base = wid * per_w
        pltpu.sync_copy(tab_hbm, tab_v)
        pltpu.sync_copy(idx_hbm.at[pl.ds(base, per_w)], idx_v)

        def body(i, carry):
            rows = idx_v[pl.ds(i * L, L)]
            r3 = rows * ND
            for d in range(ND):
                out_v[pl.ds(d * per_w + i * L, L)] = plsc.load_gather(
                    tab_v, [r3 + d])
            return carry

        lax.fori_loop(0, per_w // L, body, 0)
        pltpu.sync_copy(out_v, out_hbm.at[wid])

    return sc_gather, nw, per_w


def _mlp_kernel(pts_ref, win_ref, bin_ref, f_ref, ph_ref,
                w1_ref, b1_ref, w2_ref, b2_ref, o_ref):
    nwb = pts_ref.shape[0]
    xparts = []
    for w in range(nwb):
        p = pts_ref[w]
        xparts.append(
            lax.dot_general(p, win_ref[...], (((0,), (0,)), ((), ())))
            + jnp.sin(lax.dot_general(p, f_ref[...], (((0,), (0,)), ((), ())))
                      + ph_ref[...]))
    x = jnp.concatenate(xparts, axis=0) + bin_ref[...]
    h = jax.nn.gelu(x @ w1_ref[...] + b1_ref[...])
    y = h @ w2_ref[...] + b2_ref[...]
    o_ref[...] = jnp.mean(y.reshape(MT // K, K, HID), axis=1)


def kernel(input_pos, supernode_idxs, W_in, b_in, W1, b1, W2, b2):
    B, N, _ = input_pos.shape
    S = supernode_idxs.shape[1]
    nt = S // TS
    M = B * S * K
    si = supernode_idxs.astype(jnp.int32).reshape(B * nt, TS, 1)
    xst = jnp.transpose(input_pos, (0, 2, 1))
    F, ph = _embed_consts()

    idx = pl.pallas_call(
        _topk_kernel,
        grid=(B, nt),
        in_specs=[
            pl.BlockSpec((1, TS, 1), lambda b, j: (b * nt + j, 0, 0)),
            pl.BlockSpec((1, N, ND), lambda b, j: (b, 0, 0)),
            pl.BlockSpec((1, ND, N), lambda b, j: (b, 0, 0)),
        ],
        out_specs=pl.BlockSpec((1, TS, K), lambda b, j: (b * nt + j, 0, 0)),
        out_shape=jax.ShapeDtypeStruct((B * nt, TS, K), jnp.int32),
    )(si, input_pos, xst)
    flat_idx = idx.reshape(M)

    tab = input_pos.reshape(B * N * ND)
    sc_gather, nw, per_w = _make_sc_gather(M, B * N * ND)
    ptsw = sc_gather(tab, flat_idx)
    pw = MT // per_w
    pts3 = ptsw.reshape(nw, ND, per_w)

    out = pl.pallas_call(
        _mlp_kernel,
        grid=(M // MT,),
        in_specs=[
            pl.BlockSpec((pw, ND, per_w), lambda j: (j, 0, 0)),
            pl.BlockSpec((ND, HID), lambda j: (0, 0)),
            pl.BlockSpec((1, HID), lambda j: (0, 0)),
            pl.BlockSpec((ND, HID), lambda j: (0, 0)),
            pl.BlockSpec((1, HID), lambda j: (0, 0)),
            pl.BlockSpec((HID, HID), lambda j: (0, 0)),
            pl.BlockSpec((1, HID), lambda j: (0, 0)),
            pl.BlockSpec((HID, HID), lambda j: (0, 0)),
            pl.BlockSpec((1, HID), lambda j: (0, 0)),
        ],
        out_specs=pl.BlockSpec((MT // K, HID), lambda j: (j, 0)),
        out_shape=jax.ShapeDtypeStruct((B * S, HID), jnp.float32),
    )(pts3, W_in, b_in.reshape(1, HID), F, ph,
      W1, b1.reshape(1, HID), W2, b2.reshape(1, HID))
    return out.reshape(B, S, HID)


# speedup vs baseline: 1.0824x; 1.0212x over previous
import functools
import numpy as np
import jax
import jax.numpy as jnp
from jax import lax
from jax.experimental import pallas as pl
from jax.experimental.pallas import tpu as pltpu
from jax.experimental.pallas import tpu_sc as plsc

HID = 256
ND = 3
K = 32
TS = 128
MT = 8192


def _embed_consts():
    per = HID // ND
    half = per // 2
    emb = np.exp(np.arange(half) * -(np.log(10000.0) / (half - 1)))
    F = np.zeros((ND, HID), np.float32)
    ph = np.zeros((HID,), np.float32)
    w = 2 * half
    for i in range(ND):
        F[i, w * i: w * i + half] = emb
        F[i, w * i + half: w * i + 2 * half] = emb
        ph[w * i + half: w * i + 2 * half] = np.pi / 2
    return jnp.asarray(F), jnp.asarray(ph.reshape(1, HID))


def _topk_kernel(si_ref, xs_ref, xst_ref, o_ref):
    N = xs_ref.shape[1]
    b = pl.program_id(0)
    xs = xs_ref[0]
    xst = xst_ref[0]
    si = si_ref[0]
    iota = jax.lax.broadcasted_iota(jnp.int32, (TS, N), 1)

    q = jax.lax.dot_general((iota == si).astype(jnp.float32), xs,
                            (((1,), (0,)), ((), ())),
                            precision=jax.lax.Precision.HIGHEST)
    dist = jnp.zeros((TS, N), jnp.float32)
    for d in range(ND):
        diff = q[:, d:d + 1] - xst[d:d + 1, :]
        dist = dist + diff * diff

    idxs = []
    for _ in range(K):
        idx = jnp.argmin(dist, axis=1).astype(jnp.int32)[:, None]
        dist = jnp.where(iota == idx, jnp.inf, dist)
        idxs.append(idx)
    o_ref[0] = jnp.concatenate(idxs, axis=1) + b * N


def _make_sc_gather(n_pts, tab_len):
    info = plsc.get_sparse_core_info()
    nc, ns, L = info.num_cores, info.num_subcores, info.num_lanes
    nw = nc * ns
    per_w = n_pts // nw
    mesh = plsc.VectorSubcoreMesh(core_axis_name="c", subcore_axis_name="s")

    @functools.partial(
        pl.kernel, mesh=mesh,
        out_type=jax.ShapeDtypeStruct((nw, ND * per_w), jnp.float32),
        scratch_types=[
            pltpu.VMEM((tab_len,), jnp.float32),
            pltpu.VMEM((per_w,), jnp.int32),
            pltpu.VMEM((ND * per_w,), jnp.float32),
        ],
        compiler_params=pltpu.CompilerParams(needs_layout_passes=False),
    )
    def sc_gather(tab_hbm, idx_hbm, out_hbm, tab_v, idx_v, out_v):
        wid = lax.axis_index("s") * nc + lax.axis_index("c")
        base = wid * per_w
        pltpu.sync_copy(tab_hbm, tab_v)
        pltpu.sync_copy(idx_hbm.at[pl.ds(base, per_w)], idx_v)

        def body(i, carry):
            rows = idx_v[pl.ds(i * L, L)]
            r3 = rows * ND
            for d in range(ND):
                out_v[pl.ds(d * per_w + i * L, L)] = plsc.load_gather(
                    tab_v, [r3 + d])
            return carry

        lax.fori_loop(0, per_w // L, body, 0)
        pltpu.sync_copy(out_v, out_hbm.at[wid])

    return sc_gather, nw, per_w


def _mlp_kernel(pts_ref, win_ref, bin_ref, f_ref, ph_ref,
                w1_ref, b1_ref, w2_ref, b2_ref, o_ref):
    nwb = pts_ref.shape[0]
    xparts = []
    for w in range(nwb):
        p = pts_ref[w]
        xparts.append(
            lax.dot_general(p, win_ref[...], (((0,), (0,)), ((), ())))
            + jnp.sin(lax.dot_general(p, f_ref[...], (((0,), (0,)), ((), ())))
                      + ph_ref[...]))
    x = jnp.concatenate(xparts, axis=0) + bin_ref[...]
    h = jax.nn.gelu(x @ w1_ref[...] + b1_ref[...])
    y = h @ w2_ref[...] + b2_ref[...]
    o_ref[...] = jnp.mean(y.reshape(MT // K, K, HID), axis=1)


def kernel(input_pos, supernode_idxs, W_in, b_in, W1, b1, W2, b2):
    B, N, _ = input_pos.shape
    S = supernode_idxs.shape[1]
    nt = S // TS
    M = B * S * K
    si = supernode_idxs.astype(jnp.int32).reshape(B * nt, TS, 1)
    xst = jnp.transpose(input_pos, (0, 2, 1))
    F, ph = _embed_consts()

    idx = pl.pallas_call(
        _topk_kernel,
        grid=(B, nt),
        in_specs=[
            pl.BlockSpec((1, TS, 1), lambda b, j: (b * nt + j, 0, 0)),
            pl.BlockSpec((1, N, ND), lambda b, j: (b, 0, 0)),
            pl.BlockSpec((1, ND, N), lambda b, j: (b, 0, 0)),
        ],
        out_specs=pl.BlockSpec((1, TS, K), lambda b, j: (b * nt + j, 0, 0)),
        out_shape=jax.ShapeDtypeStruct((B * nt, TS, K), jnp.int32),
    )(si, input_pos, xst)
    flat_idx = idx.reshape(M)

    tab = input_pos.reshape(B * N * ND)
    sc_gather, nw, per_w = _make_sc_gather(M, B * N * ND)
    ptsw = sc_gather(tab, flat_idx)
    pw = MT // per_w
    pts3 = ptsw.reshape(nw, ND, per_w)

    out = pl.pallas_call(
        _mlp_kernel,
        grid=(M // MT,),
        in_specs=[
            pl.BlockSpec((pw, ND, per_w), lambda j: (j, 0, 0)),
            pl.BlockSpec((ND, HID), lambda j: (0, 0)),
            pl.BlockSpec((1, HID), lambda j: (0, 0)),
            pl.BlockSpec((ND, HID), lambda j: (0, 0)),
            pl.BlockSpec((1, HID), lambda j: (0, 0)),
            pl.BlockSpec((HID, HID), lambda j: (0, 0)),
            pl.BlockSpec((1, HID), lambda j: (0, 0)),
            pl.BlockSpec((HID, HID), lambda j: (0, 0)),
            pl.BlockSpec((1, HID), lambda j: (0, 0)),
        ],
        out_specs=pl.BlockSpec((MT // K, HID), lambda j: (j, 0)),
        out_shape=jax.ShapeDtypeStruct((B * S, HID), jnp.float32),
    )(pts3, W_in, b_in.reshape(1, HID), F, ph,
      W1, b1.reshape(1, HID), W2, b2.reshape(1, HID))
    return out.reshape(B, S, HID)
